# Initial kernel scaffold; baseline (speedup 1.0000x reference)
import jax
import jax.numpy as jnp
from jax.experimental import pallas as pl


def kernel(x, adj, edges, W1, b1, W2, b2, W3, b3):
    raise NotImplementedError("write your pallas kernel here")


# speedup vs baseline: 4.0673x; 4.0673x over previous
import functools

import jax
import jax.numpy as jnp
from jax import lax
from jax.experimental import pallas as pl
from jax.experimental.pallas import tpu as pltpu
from jax.experimental.pallas import tpu_sc as plsc

_N = 4096
_E = 4096
_C = 128
_H = 256
_THRESH = 0.996
_BLK = 256
_NB = _N // _BLK
_W = _N // 32


def _pack_bits(a_bool, rows):
    ai = a_bool.astype(jnp.int32)
    acc = jnp.zeros((rows, _W), jnp.int32)
    for b in range(32):
        acc = acc | (ai[:, b * _W:(b + 1) * _W] << b)
    return acc


def _build_a_body(adj_row_ref, adj_col_ref, a_bf_ref, a_bits_ref):
    i = pl.program_id(0)
    r = adj_row_ref[...] > _THRESH
    ct = jnp.transpose(adj_col_ref[...]) > _THRESH
    a = r | ct
    row_g = i * _BLK + lax.broadcasted_iota(jnp.int32, (_BLK, _N), 0)
    col_g = lax.broadcasted_iota(jnp.int32, (_BLK, _N), 1)
    a = a & (row_g != col_g)
    a_bf_ref[...] = a.astype(jnp.bfloat16)
    a_bits_ref[...] = _pack_bits(a, _BLK)


def _build_a(adj):
    return pl.pallas_call(
        _build_a_body,
        grid=(_NB,),
        in_specs=[
            pl.BlockSpec((_BLK, _N), lambda i: (i, 0)),
            pl.BlockSpec((_N, _BLK), lambda i: (0, i)),
        ],
        out_specs=[
            pl.BlockSpec((_BLK, _N), lambda i: (i, 0)),
            pl.BlockSpec((_BLK, _W), lambda i: (i, 0)),
        ],
        out_shape=[
            jax.ShapeDtypeStruct((_N, _N), jnp.bfloat16),
            jax.ShapeDtypeStruct((_N, _W), jnp.int32),
        ],
    )(adj, adj)


def _a2_body(panel_ref, full_ref, a2_bits_ref):
    i = pl.program_id(0)
    p = panel_ref[...]
    m = jnp.dot(p, full_ref[...], preferred_element_type=jnp.float32)
    row_g = i * _BLK + lax.broadcasted_iota(jnp.int32, (_BLK, _N), 0)
    col_g = lax.broadcasted_iota(jnp.int32, (_BLK, _N), 1)
    a2 = (m > 0.5) & (p == 0) & (row_g != col_g)
    a2_bits_ref[...] = _pack_bits(a2, _BLK)


def _build_a2(a_bf):
    return pl.pallas_call(
        _a2_body,
        grid=(_NB,),
        in_specs=[
            pl.BlockSpec((_BLK, _N), lambda i: (i, 0)),
            pl.BlockSpec((_N, _N), lambda i: (0, 0)),
        ],
        out_specs=pl.BlockSpec((_BLK, _W), lambda i: (i, 0)),
        out_shape=jax.ShapeDtypeStruct((_N, _W), jnp.int32),
        compiler_params=pltpu.CompilerParams(
            vmem_limit_bytes=100 * 1024 * 1024),
    )(a_bf, a_bf)


def _gather_rows(a_bits, a2_bits, x, src, dst):
    mesh = plsc.VectorSubcoreMesh(core_axis_name="c", subcore_axis_name="s")
    nw = 32
    bw = _E // nw

    @functools.partial(
        pl.kernel,
        mesh=mesh,
        out_type=[jax.ShapeDtypeStruct((_E, _W), jnp.int32)] * 4
        + [jax.ShapeDtypeStruct((_E, _C), jnp.float32)] * 2,
        scratch_types=[
            pltpu.VMEM((bw,), jnp.int32),
            pltpu.VMEM((bw,), jnp.int32),
            pltpu.VMEM((bw, _W), jnp.int32),
            pltpu.VMEM((bw, _C), jnp.float32),
            pltpu.SemaphoreType.DMA,
        ],
    )
    def k(ab_hbm, a2b_hbm, x_hbm, src_hbm, dst_hbm,
          au_o, av_o, a2u_o, a2v_o, xi_o, xj_o,
          idx_u, idx_v, bi, bf, sem):
        wid = lax.axis_index("s") * 2 + lax.axis_index("c")
        base = wid * bw
        pltpu.sync_copy(src_hbm.at[pl.ds(base, bw)], idx_u)
        pltpu.sync_copy(dst_hbm.at[pl.ds(base, bw)], idx_v)
        for table, idx, buf, out in (
            (ab_hbm, idx_u, bi, au_o),
            (ab_hbm, idx_v, bi, av_o),
            (a2b_hbm, idx_u, bi, a2u_o),
            (a2b_hbm, idx_v, bi, a2v_o),
            (x_hbm, idx_u, bf, xi_o),
            (x_hbm, idx_v, bf, xj_o),
        ):
            pltpu.async_copy(table.at[idx], buf, sem).wait()
            pltpu.sync_copy(buf, out.at[pl.ds(base, bw)])

    return k(a_bits, a2_bits, x, src, dst)


def _popcount(v):
    v = v - (lax.shift_right_logical(v, 1) & 0x55555555)
    v = (v & 0x33333333) + (lax.shift_right_logical(v, 2) & 0x33333333)
    v = (v + lax.shift_right_logical(v, 4)) & 0x0F0F0F0F
    v = v + lax.shift_right_logical(v, 8)
    v = v + lax.shift_right_logical(v, 16)
    return v & 0x3F


def _dot3(a, b, dn):
    ah = a.astype(jnp.bfloat16)
    al = (a - ah.astype(jnp.float32)).astype(jnp.bfloat16)
    bh = b.astype(jnp.bfloat16)
    bl = (b - bh.astype(jnp.float32)).astype(jnp.bfloat16)
    r = lax.dot_general(ah, bh, dn, preferred_element_type=jnp.float32)
    r = r + lax.dot_general(ah, bl, dn, preferred_element_type=jnp.float32)
    r = r + lax.dot_general(al, bh, dn, preferred_element_type=jnp.float32)
    return r


def _counts_body(au_ref, av_ref, a2u_ref, a2v_ref, dstb_ref, out_ref):
    au = au_ref[...]
    av = av_ref[...]
    a2u = a2u_ref[...]
    a2v = a2v_ref[...]

    def pcsum(v):
        return jnp.sum(_popcount(v).astype(jnp.float32), axis=1, keepdims=True)

    c11 = pcsum(au & av)
    c12 = pcsum(au & a2v) + pcsum(a2u & av)
    c22 = pcsum(a2u & a2v)
    degu = pcsum(au)
    degv = pcsum(av)
    deg2u = pcsum(a2u)
    deg2v = pcsum(a2v)

    dstv = dstb_ref[...]
    lane = lax.broadcasted_iota(jnp.int32, (_E, _W), 1)
    mask = jnp.where(lane == (dstv & 127),
                     lax.shift_left(jnp.ones_like(dstv),
                                    lax.shift_right_logical(dstv, 7)),
                     0)
    auv = jnp.sum(((au & mask) != 0).astype(jnp.float32), axis=1, keepdims=True)
    a2uv = jnp.sum(((a2u & mask) != 0).astype(jnp.float32), axis=1,
                   keepdims=True)

    f1 = c11
    f2 = c12
    f3 = degu + degv - 2.0 * auv - 2.0 * c11 - c12
    f4 = c22
    f5 = deg2u + deg2v - 2.0 * a2uv - 2.0 * c22 - c12
    one = jnp.ones_like(f1)
    pad = jnp.zeros_like(f1)
    out_ref[...] = jnp.concatenate([f1, f2, f3, f4, f5, one, pad, pad], axis=1)


def _counts(au, av, a2u, a2v, dstb):
    return pl.pallas_call(
        _counts_body,
        out_shape=jax.ShapeDtypeStruct((_E, 8), jnp.float32),
    )(au, av, a2u, a2v, dstb)


def _mlp_body(z_ref, xi_ref, xj_ref, w1a_ref, w1bp_ref,
              w2_ref, b2b_ref, w3b_ref, out_ref):
    dn1 = (((1,), (1,)), ((), ()))
    xx = xi_ref[...] * xj_ref[...]
    h = lax.dot_general(xx, w1a_ref[...], dn1,
                        preferred_element_type=jnp.float32)
    h = h + lax.dot_general(z_ref[...], w1bp_ref[...], (((1,), (0,)), ((), ())),
                            preferred_element_type=jnp.float32)
    h = jnp.maximum(h, 0.0)
    h = lax.dot_general(h, w2_ref[...], dn1,
                        preferred_element_type=jnp.float32)
    h = jnp.maximum(h + b2b_ref[...], 0.0)
    hb = h.astype(jnp.bfloat16).astype(jnp.float32)
    w3r = w3b_ref[...].astype(jnp.bfloat16).astype(jnp.float32)
    logit = jnp.sum(hb * w3r, axis=1, keepdims=True)
    pad = jnp.zeros_like(logit)
    out_ref[...] = jnp.concatenate([logit] + [pad] * 7, axis=1)


def _mlp(z, xi, xj, w1a, w1bp, w2, b2b, w3b):
    return pl.pallas_call(
        _mlp_body,
        out_shape=jax.ShapeDtypeStruct((_E, 8), jnp.float32),
    )(z, xi, xj, w1a, w1bp, w2, b2b, w3b)


def kernel(x, adj, edges, W1, b1, W2, b2, W3, b3):
    src = edges[0]
    dst = edges[1]
    a_bf, a_bits = _build_a(adj)
    a2_bits = _build_a2(a_bf)
    au, av, a2u, a2v, xi, xj = _gather_rows(a_bits, a2_bits, x, src, dst)
    dstb = jnp.broadcast_to(dst[:, None], (_E, _W))
    z = _counts(au, av, a2u, a2v, dstb)
    w1a = W1[:, :_C]
    w1bp = jnp.concatenate(
        [jnp.transpose(W1[:, _C:]), b1[None, :],
         jnp.zeros((2, _H), jnp.float32)], axis=0)
    b2b = jnp.broadcast_to(b2[None, :], (_E, _H))
    w3b = jnp.broadcast_to(W3, (_E, _H))
    out = _mlp(z, xi, xj, w1a, w1bp, w2=W2, b2b=b2b, w3b=w3b)
    return out[:, :1] + b3


# speedup vs baseline: 5.7378x; 1.4107x over previous
import functools

import jax
import jax.numpy as jnp
from jax import lax
from jax.experimental import pallas as pl
from jax.experimental.pallas import tpu as pltpu
from jax.experimental.pallas import tpu_sc as plsc

_N = 4096
_E = 4096
_C = 128
_H = 256
_THRESH = 0.996
_BLK = 256
_NB = _N // _BLK
_W = _N // 32


def _pack_bits(a_bool, rows):
    ai = a_bool.astype(jnp.int32)
    acc = jnp.zeros((rows, _W), jnp.int32)
    for b in range(32):
        acc = acc | (ai[:, b * _W:(b + 1) * _W] << b)
    return acc


def _build_a_body(adj_row_ref, adj_col_ref, a_bf_ref, a_bits_ref):
    i = pl.program_id(0)
    r = adj_row_ref[...] > _THRESH
    ct = jnp.transpose(adj_col_ref[...]) > _THRESH
    a = r | ct
    row_g = i * _BLK + lax.broadcasted_iota(jnp.int32, (_BLK, _N), 0)
    col_g = lax.broadcasted_iota(jnp.int32, (_BLK, _N), 1)
    a = a & (row_g != col_g)
    a_bf_ref[...] = a.astype(jnp.float8_e4m3fn)
    a_bits_ref[...] = _pack_bits(a, _BLK)


def _build_a(adj):
    return pl.pallas_call(
        _build_a_body,
        grid=(_NB,),
        in_specs=[
            pl.BlockSpec((_BLK, _N), lambda i: (i, 0)),
            pl.BlockSpec((_N, _BLK), lambda i: (0, i)),
        ],
        out_specs=[
            pl.BlockSpec((_BLK, _N), lambda i: (i, 0)),
            pl.BlockSpec((_BLK, _W), lambda i: (i, 0)),
        ],
        out_shape=[
            jax.ShapeDtypeStruct((_N, _N), jnp.float8_e4m3fn),
            jax.ShapeDtypeStruct((_N, _W), jnp.int32),
        ],
    )(adj, adj)


def _a2_body(panel_ref, full_ref, a2_bits_ref):
    i = pl.program_id(0)
    p = panel_ref[...]
    m = jnp.dot(p, full_ref[...], preferred_element_type=jnp.float32)
    row_g = i * _BLK + lax.broadcasted_iota(jnp.int32, (_BLK, _N), 0)
    col_g = lax.broadcasted_iota(jnp.int32, (_BLK, _N), 1)
    a2 = (m > 0.5) & (p.astype(jnp.float32) == 0) & (row_g != col_g)
    a2_bits_ref[...] = _pack_bits(a2, _BLK)


def _build_a2(a_bf):
    return pl.pallas_call(
        _a2_body,
        grid=(_NB,),
        in_specs=[
            pl.BlockSpec((_BLK, _N), lambda i: (i, 0)),
            pl.BlockSpec((_N, _N), lambda i: (0, 0)),
        ],
        out_specs=pl.BlockSpec((_BLK, _W), lambda i: (i, 0)),
        out_shape=jax.ShapeDtypeStruct((_N, _W), jnp.int32),
        compiler_params=pltpu.CompilerParams(
            vmem_limit_bytes=100 * 1024 * 1024),
    )(a_bf, a_bf)


def _gather_rows(a_bits, a2_bits, x, src, dst):
    mesh = plsc.VectorSubcoreMesh(core_axis_name="c", subcore_axis_name="s")
    nw = 32
    bw = _E // nw

    @functools.partial(
        pl.kernel,
        mesh=mesh,
        out_type=[jax.ShapeDtypeStruct((_E, _W), jnp.int32)] * 4
        + [jax.ShapeDtypeStruct((_E, _C), jnp.float32)] * 2,
        scratch_types=[
            pltpu.VMEM((bw,), jnp.int32),
            pltpu.VMEM((bw,), jnp.int32),
            pltpu.VMEM((bw, _W), jnp.int32),
            pltpu.VMEM((bw, _C), jnp.float32),
            pltpu.SemaphoreType.DMA,
        ],
    )
    def k(ab_hbm, a2b_hbm, x_hbm, src_hbm, dst_hbm,
          au_o, av_o, a2u_o, a2v_o, xi_o, xj_o,
          idx_u, idx_v, bi, bf, sem):
        wid = lax.axis_index("s") * 2 + lax.axis_index("c")
        base = wid * bw
        pltpu.sync_copy(src_hbm.at[pl.ds(base, bw)], idx_u)
        pltpu.sync_copy(dst_hbm.at[pl.ds(base, bw)], idx_v)
        for table, idx, buf, out in (
            (ab_hbm, idx_u, bi, au_o),
            (ab_hbm, idx_v, bi, av_o),
            (a2b_hbm, idx_u, bi, a2u_o),
            (a2b_hbm, idx_v, bi, a2v_o),
            (x_hbm, idx_u, bf, xi_o),
            (x_hbm, idx_v, bf, xj_o),
        ):
            pltpu.async_copy(table.at[idx], buf, sem).wait()
            pltpu.sync_copy(buf, out.at[pl.ds(base, bw)])

    return k(a_bits, a2_bits, x, src, dst)


def _popcount(v):
    v = v - (lax.shift_right_logical(v, 1) & 0x55555555)
    v = (v & 0x33333333) + (lax.shift_right_logical(v, 2) & 0x33333333)
    v = (v + lax.shift_right_logical(v, 4)) & 0x0F0F0F0F
    v = v + lax.shift_right_logical(v, 8)
    v = v + lax.shift_right_logical(v, 16)
    return v & 0x3F


def _dot3(a, b, dn):
    ah = a.astype(jnp.bfloat16)
    al = (a - ah.astype(jnp.float32)).astype(jnp.bfloat16)
    bh = b.astype(jnp.bfloat16)
    bl = (b - bh.astype(jnp.float32)).astype(jnp.bfloat16)
    r = lax.dot_general(ah, bh, dn, preferred_element_type=jnp.float32)
    r = r + lax.dot_general(ah, bl, dn, preferred_element_type=jnp.float32)
    r = r + lax.dot_general(al, bh, dn, preferred_element_type=jnp.float32)
    return r


def _counts_body(au_ref, av_ref, a2u_ref, a2v_ref, dstb_ref, out_ref):
    au = au_ref[...]
    av = av_ref[...]
    a2u = a2u_ref[...]
    a2v = a2v_ref[...]

    def pcsum(v):
        return jnp.sum(_popcount(v).astype(jnp.float32), axis=1, keepdims=True)

    c11 = pcsum(au & av)
    c12 = pcsum(au & a2v) + pcsum(a2u & av)
    c22 = pcsum(a2u & a2v)
    degu = pcsum(au)
    degv = pcsum(av)
    deg2u = pcsum(a2u)
    deg2v = pcsum(a2v)

    dstv = dstb_ref[...]
    lane = lax.broadcasted_iota(jnp.int32, (_E, _W), 1)
    mask = jnp.where(lane == (dstv & 127),
                     lax.shift_left(jnp.ones_like(dstv),
                                    lax.shift_right_logical(dstv, 7)),
                     0)
    auv = jnp.sum(((au & mask) != 0).astype(jnp.float32), axis=1, keepdims=True)
    a2uv = jnp.sum(((a2u & mask) != 0).astype(jnp.float32), axis=1,
                   keepdims=True)

    f1 = c11
    f2 = c12
    f3 = degu + degv - 2.0 * auv - 2.0 * c11 - c12
    f4 = c22
    f5 = deg2u + deg2v - 2.0 * a2uv - 2.0 * c22 - c12
    one = jnp.ones_like(f1)
    pad = jnp.zeros_like(f1)
    out_ref[...] = jnp.concatenate([f1, f2, f3, f4, f5, one, pad, pad], axis=1)


def _counts(au, av, a2u, a2v, dstb):
    return pl.pallas_call(
        _counts_body,
        out_shape=jax.ShapeDtypeStruct((_E, 8), jnp.float32),
    )(au, av, a2u, a2v, dstb)


def _mlp_body(z_ref, xi_ref, xj_ref, w1a_ref, w1bp_ref,
              w2_ref, b2b_ref, w3b_ref, out_ref):
    dn1 = (((1,), (1,)), ((), ()))
    xx = xi_ref[...] * xj_ref[...]
    h = lax.dot_general(xx, w1a_ref[...], dn1,
                        preferred_element_type=jnp.float32)
    h = h + lax.dot_general(z_ref[...], w1bp_ref[...], (((1,), (0,)), ((), ())),
                            preferred_element_type=jnp.float32)
    h = jnp.maximum(h, 0.0)
    h = lax.dot_general(h, w2_ref[...], dn1,
                        preferred_element_type=jnp.float32)
    h = jnp.maximum(h + b2b_ref[...], 0.0)
    hb = h.astype(jnp.bfloat16).astype(jnp.float32)
    w3r = w3b_ref[...].astype(jnp.bfloat16).astype(jnp.float32)
    logit = jnp.sum(hb * w3r, axis=1, keepdims=True)
    pad = jnp.zeros_like(logit)
    out_ref[...] = jnp.concatenate([logit] + [pad] * 7, axis=1)


def _mlp(z, xi, xj, w1a, w1bp, w2, b2b, w3b):
    return pl.pallas_call(
        _mlp_body,
        out_shape=jax.ShapeDtypeStruct((_E, 8), jnp.float32),
    )(z, xi, xj, w1a, w1bp, w2, b2b, w3b)


def kernel(x, adj, edges, W1, b1, W2, b2, W3, b3):
    src = edges[0]
    dst = edges[1]
    a_bf, a_bits = _build_a(adj)
    a2_bits = _build_a2(a_bf)
    au, av, a2u, a2v, xi, xj = _gather_rows(a_bits, a2_bits, x, src, dst)
    dstb = jnp.broadcast_to(dst[:, None], (_E, _W))
    z = _counts(au, av, a2u, a2v, dstb)
    w1a = W1[:, :_C]
    w1bp = jnp.concatenate(
        [jnp.transpose(W1[:, _C:]), b1[None, :],
         jnp.zeros((2, _H), jnp.float32)], axis=0)
    b2b = jnp.broadcast_to(b2[None, :], (_E, _H))
    w3b = jnp.broadcast_to(W3, (_E, _H))
    out = _mlp(z, xi, xj, w1a, w1bp, w2=W2, b2b=b2b, w3b=w3b)
    return out[:, :1] + b3


# speedup vs baseline: 6.0790x; 1.0595x over previous
import functools

import jax
import jax.numpy as jnp
from jax import lax
from jax.experimental import pallas as pl
from jax.experimental.pallas import tpu as pltpu
from jax.experimental.pallas import tpu_sc as plsc

_N = 4096
_E = 4096
_C = 128
_H = 256
_THRESH = 0.996
_BLK = 256
_NB = _N // _BLK
_W = _N // 32


def _pack_bits(a_bool, rows):
    ai = a_bool.astype(jnp.int32)
    acc = jnp.zeros((rows, _W), jnp.int32)
    for b in range(32):
        acc = acc | (ai[:, b * _W:(b + 1) * _W] << b)
    return acc


def _build_a_body(adj_row_ref, adj_col_ref, a_bf_ref, a_bits_ref):
    i = pl.program_id(0)
    r = adj_row_ref[...] > _THRESH
    ct = jnp.transpose(adj_col_ref[...]) > _THRESH
    a = r | ct
    row_g = i * _BLK + lax.broadcasted_iota(jnp.int32, (_BLK, _N), 0)
    col_g = lax.broadcasted_iota(jnp.int32, (_BLK, _N), 1)
    a = a & (row_g != col_g)
    a_bf_ref[...] = a.astype(jnp.float8_e4m3fn)
    a_bits_ref[...] = _pack_bits(a, _BLK)


def _build_a(adj):
    return pl.pallas_call(
        _build_a_body,
        grid=(_NB,),
        in_specs=[
            pl.BlockSpec((_BLK, _N), lambda i: (i, 0)),
            pl.BlockSpec((_N, _BLK), lambda i: (0, i)),
        ],
        out_specs=[
            pl.BlockSpec((_BLK, _N), lambda i: (i, 0)),
            pl.BlockSpec((_BLK, _W), lambda i: (i, 0)),
        ],
        out_shape=[
            jax.ShapeDtypeStruct((_N, _N), jnp.float8_e4m3fn),
            jax.ShapeDtypeStruct((_N, _W), jnp.int32),
        ],
    )(adj, adj)


def _a2_body(panel_ref, full_ref, a2_bits_ref):
    i = pl.program_id(0)
    p = panel_ref[...]
    m = jnp.dot(p, full_ref[...], preferred_element_type=jnp.float32)
    row_g = i * _BLK + lax.broadcasted_iota(jnp.int32, (_BLK, _N), 0)
    col_g = lax.broadcasted_iota(jnp.int32, (_BLK, _N), 1)
    a2 = (m > 0.5) & (p.astype(jnp.float32) == 0) & (row_g != col_g)
    a2_bits_ref[...] = _pack_bits(a2, _BLK)


def _build_a2(a_bf):
    return pl.pallas_call(
        _a2_body,
        grid=(_NB,),
        in_specs=[
            pl.BlockSpec((_BLK, _N), lambda i: (i, 0)),
            pl.BlockSpec((_N, _N), lambda i: (0, 0)),
        ],
        out_specs=pl.BlockSpec((_BLK, _W), lambda i: (i, 0)),
        out_shape=jax.ShapeDtypeStruct((_N, _W), jnp.int32),
        compiler_params=pltpu.CompilerParams(
            vmem_limit_bytes=100 * 1024 * 1024),
    )(a_bf, a_bf)


def _gather_first(a_bits, x, src, dst):
    mesh = plsc.VectorSubcoreMesh(core_axis_name="c", subcore_axis_name="s")
    nw = 32
    bw = _E // nw

    @functools.partial(
        pl.kernel,
        mesh=mesh,
        out_type=[jax.ShapeDtypeStruct((_E, _W), jnp.int32)] * 2
        + [jax.ShapeDtypeStruct((_E, _C), jnp.float32)] * 2,
        scratch_types=[
            pltpu.VMEM((bw,), jnp.int32),
            pltpu.VMEM((bw,), jnp.int32),
            pltpu.VMEM((bw, _W), jnp.int32),
            pltpu.VMEM((bw, _C), jnp.float32),
            pltpu.SemaphoreType.DMA,
        ],
    )
    def k(ab_hbm, x_hbm, src_hbm, dst_hbm,
          au_o, av_o, xi_o, xj_o,
          idx_u, idx_v, bi, bf, sem):
        wid = lax.axis_index("s") * 2 + lax.axis_index("c")
        base = wid * bw
        pltpu.sync_copy(src_hbm.at[pl.ds(base, bw)], idx_u)
        pltpu.sync_copy(dst_hbm.at[pl.ds(base, bw)], idx_v)
        for table, idx, buf, out in (
            (ab_hbm, idx_u, bi, au_o),
            (ab_hbm, idx_v, bi, av_o),
            (x_hbm, idx_u, bf, xi_o),
            (x_hbm, idx_v, bf, xj_o),
        ):
            pltpu.async_copy(table.at[idx], buf, sem).wait()
            pltpu.sync_copy(buf, out.at[pl.ds(base, bw)])

    return k(a_bits, x, src, dst)


def _gather_second(a2_bits, src, dst):
    mesh = plsc.VectorSubcoreMesh(core_axis_name="c", subcore_axis_name="s")
    nw = 32
    bw = _E // nw

    @functools.partial(
        pl.kernel,
        mesh=mesh,
        out_type=[jax.ShapeDtypeStruct((_E, _W), jnp.int32)] * 2,
        scratch_types=[
            pltpu.VMEM((bw,), jnp.int32),
            pltpu.VMEM((bw,), jnp.int32),
            pltpu.VMEM((bw, _W), jnp.int32),
            pltpu.SemaphoreType.DMA,
        ],
    )
    def k(a2b_hbm, src_hbm, dst_hbm, a2u_o, a2v_o, idx_u, idx_v, bi, sem):
        wid = lax.axis_index("s") * 2 + lax.axis_index("c")
        base = wid * bw
        pltpu.sync_copy(src_hbm.at[pl.ds(base, bw)], idx_u)
        pltpu.sync_copy(dst_hbm.at[pl.ds(base, bw)], idx_v)
        for table, idx, buf, out in (
            (a2b_hbm, idx_u, bi, a2u_o),
            (a2b_hbm, idx_v, bi, a2v_o),
        ):
            pltpu.async_copy(table.at[idx], buf, sem).wait()
            pltpu.sync_copy(buf, out.at[pl.ds(base, bw)])

    return k(a2_bits, src, dst)


def _popcount(v):
    v = v - (lax.shift_right_logical(v, 1) & 0x55555555)
    v = (v & 0x33333333) + (lax.shift_right_logical(v, 2) & 0x33333333)
    v = (v + lax.shift_right_logical(v, 4)) & 0x0F0F0F0F
    v = v + lax.shift_right_logical(v, 8)
    v = v + lax.shift_right_logical(v, 16)
    return v & 0x3F


def _dot3(a, b, dn):
    ah = a.astype(jnp.bfloat16)
    al = (a - ah.astype(jnp.float32)).astype(jnp.bfloat16)
    bh = b.astype(jnp.bfloat16)
    bl = (b - bh.astype(jnp.float32)).astype(jnp.bfloat16)
    r = lax.dot_general(ah, bh, dn, preferred_element_type=jnp.float32)
    r = r + lax.dot_general(ah, bl, dn, preferred_element_type=jnp.float32)
    r = r + lax.dot_general(al, bh, dn, preferred_element_type=jnp.float32)
    return r


def _mlp_body(au_ref, av_ref, a2u_ref, a2v_ref, dstb_ref,
              xi_ref, xj_ref, w1a_ref, w1bp_ref,
              w2_ref, b2b_ref, w3b_ref, out_ref):
    au = au_ref[...]
    av = av_ref[...]
    a2u = a2u_ref[...]
    a2v = a2v_ref[...]

    def pcsum(v):
        return jnp.sum(_popcount(v).astype(jnp.float32), axis=1, keepdims=True)

    c11 = pcsum(au & av)
    c12 = pcsum(au & a2v) + pcsum(a2u & av)
    c22 = pcsum(a2u & a2v)
    degu = pcsum(au)
    degv = pcsum(av)
    deg2u = pcsum(a2u)
    deg2v = pcsum(a2v)

    dstv = dstb_ref[...]
    lane = lax.broadcasted_iota(jnp.int32, (_E, _W), 1)
    mask = jnp.where(lane == (dstv & 127),
                     lax.shift_left(jnp.ones_like(dstv),
                                    lax.shift_right_logical(dstv, 7)),
                     0)
    auv = jnp.sum(((au & mask) != 0).astype(jnp.float32), axis=1, keepdims=True)
    a2uv = jnp.sum(((a2u & mask) != 0).astype(jnp.float32), axis=1,
                   keepdims=True)

    f1 = c11
    f2 = c12
    f3 = degu + degv - 2.0 * auv - 2.0 * c11 - c12
    f4 = c22
    f5 = deg2u + deg2v - 2.0 * a2uv - 2.0 * c22 - c12
    one = jnp.ones_like(f1)
    pad = jnp.zeros_like(f1)
    z = jnp.concatenate([f1, f2, f3, f4, f5, one, pad, pad], axis=1)

    dn1 = (((1,), (1,)), ((), ()))
    xx = xi_ref[...] * xj_ref[...]
    h = lax.dot_general(xx, w1a_ref[...], dn1,
                        preferred_element_type=jnp.float32)
    h = h + lax.dot_general(z, w1bp_ref[...], (((1,), (0,)), ((), ())),
                            preferred_element_type=jnp.float32)
    h = jnp.maximum(h, 0.0)
    h = lax.dot_general(h, w2_ref[...], dn1,
                        preferred_element_type=jnp.float32)
    h = jnp.maximum(h + b2b_ref[...], 0.0)
    hb = h.astype(jnp.bfloat16).astype(jnp.float32)
    w3r = w3b_ref[...].astype(jnp.bfloat16).astype(jnp.float32)
    logit = jnp.sum(hb * w3r, axis=1, keepdims=True)
    pad2 = jnp.zeros_like(logit)
    out_ref[...] = jnp.concatenate([logit] + [pad2] * 7, axis=1)


def _mlp(au, av, a2u, a2v, dstb, xi, xj, w1a, w1bp, w2, b2b, w3b):
    return pl.pallas_call(
        _mlp_body,
        out_shape=jax.ShapeDtypeStruct((_E, 8), jnp.float32),
    )(au, av, a2u, a2v, dstb, xi, xj, w1a, w1bp, w2, b2b, w3b)


def kernel(x, adj, edges, W1, b1, W2, b2, W3, b3):
    src = edges[0]
    dst = edges[1]
    a_f8, a_bits = _build_a(adj)
    au, av, xi, xj = _gather_first(a_bits, x, src, dst)
    a2_bits = _build_a2(a_f8)
    a2u, a2v = _gather_second(a2_bits, src, dst)
    dstb = jnp.broadcast_to(dst[:, None], (_E, _W))
    w1a = W1[:, :_C]
    w1bp = jnp.concatenate(
        [jnp.transpose(W1[:, _C:]), b1[None, :],
         jnp.zeros((2, _H), jnp.float32)], axis=0)
    b2b = jnp.broadcast_to(b2[None, :], (_E, _H))
    w3b = jnp.broadcast_to(W3, (_E, _H))
    out = _mlp(au, av, a2u, a2v, dstb, xi, xj, w1a, w1bp, W2, b2b, w3b)
    return out[:, :1] + b3
